# baseline (speedup 1.0000x reference)
import functools
import math

import jax
import jax.numpy as jnp
from jax import lax
from jax.experimental import pallas as pl
from jax.experimental.pallas import tpu as pltpu
from jax.experimental.pallas import tpu_sc as plsc

_NUM_WORKERS = 16
_CHUNK = 512
_LANES = 16


def _emb_lookup(flat_idx, table, *, b_dim, l_dim, d, scale):
    mesh = plsc.VectorSubcoreMesh(core_axis_name="c", subcore_axis_name="s", num_cores=1)
    n = b_dim * l_dim
    per_worker = n // _NUM_WORKERS
    nc = per_worker // _CHUNK
    assert nc % 2 == 0 and nc >= 4
    assert b_dim % _CHUNK == 0 and (b_dim & (b_dim - 1)) == 0
    b_shift = b_dim.bit_length() - 1

    @functools.partial(
        pl.kernel,
        mesh=mesh,
        out_type=jax.ShapeDtypeStruct((l_dim, d, b_dim), jnp.float32),
        scratch_types=[
            pltpu.VMEM((per_worker,), jnp.int32),
            [pltpu.VMEM((_CHUNK, d), jnp.float32) for _ in range(2)],
            [pltpu.VMEM((d, _CHUNK + 1), jnp.float32) for _ in range(2)],
            [pltpu.SemaphoreType.DMA for _ in range(2)],
            [pltpu.SemaphoreType.DMA for _ in range(2)],
        ],
        compiler_params=pltpu.CompilerParams(
            use_tc_tiling_on_sc=False, needs_layout_passes=False),
    )
    def body(idx_hbm, table_hbm, out_hbm, idx_v, rows_v, t_v, gsem, wsem):
        wid = lax.axis_index("s")
        base = wid * per_worker
        pltpu.sync_copy(idx_hbm.at[pl.ds(base, per_worker)], idx_v)

        def start_gather(ci, b):
            off = pl.multiple_of(ci * _CHUNK, 8)
            pltpu.async_copy(
                table_hbm.at[idx_v.at[pl.ds(off, _CHUNK)]], rows_v[b], gsem[b])

        def wait_gather(b):
            pltpu.make_async_copy(
                table_hbm.at[idx_v.at[pl.ds(0, _CHUNK)]], rows_v[b],
                gsem[b]).wait()

        def wait_writeback(b):
            pltpu.make_async_copy(
                t_v[b].at[:, pl.ds(0, _CHUNK)],
                out_hbm.at[0, :, pl.ds(0, _CHUNK)], wsem[b]).wait()

        start_gather(0, 0)
        start_gather(1, 1)

        @pl.loop(0, nc // 2)
        def pair_loop(p):
            for b in range(2):
                ci = p * 2 + b
                wait_gather(b)

                @pl.when(p > 0)
                def _():
                    wait_writeback(b)

                jlo = lax.iota(jnp.int32, _LANES)
                jhi = jlo + _LANES

                def transpose4(r4, carry, rows=rows_v[b], t=t_v[b]):
                    for k in range(4):
                        r = r4 * 4 + k
                        rvec = jnp.full((_LANES,), r, jnp.int32)
                        plsc.store_scatter(
                            t, [jlo, rvec], rows[r, pl.ds(0, _LANES)] * scale)
                        plsc.store_scatter(
                            t, [jhi, rvec],
                            rows[r, pl.ds(_LANES, _LANES)] * scale)
                    return carry

                lax.fori_loop(0, _CHUNK // 4, transpose4, 0)
                gpos = base + ci * _CHUNK
                l_i = lax.shift_right_logical(gpos, b_shift)
                b0 = pl.multiple_of(gpos & (b_dim - 1), _CHUNK)
                pltpu.async_copy(
                    t_v[b].at[:, pl.ds(0, _CHUNK)],
                    out_hbm.at[l_i, :, pl.ds(b0, _CHUNK)], wsem[b])

                @pl.when(ci + 2 < nc)
                def _():
                    start_gather(ci + 2, b)

        for b in range(2):
            wait_writeback(b)

    return body(flat_idx, table)


def kernel(tokens, table):
    b_dim, l_dim = tokens.shape
    v, d = table.shape
    flat_idx = tokens.T.reshape(b_dim * l_dim).astype(jnp.int32)
    out_nat = _emb_lookup(
        flat_idx,
        table,
        b_dim=b_dim,
        l_dim=l_dim,
        d=d,
        scale=math.sqrt(d),
    )
    return jnp.transpose(out_nat, (2, 0, 1))


# speedup vs baseline: 1.2879x; 1.2879x over previous
import functools
import math

import jax
import jax.numpy as jnp
from jax import lax
from jax.experimental import pallas as pl
from jax.experimental.pallas import tpu as pltpu
from jax.experimental.pallas import tpu_sc as plsc

_NUM_WORKERS = 32
_CHUNK = 512
_LANES = 16


def _emb_lookup(flat_idx, table, *, b_dim, l_dim, d, scale):
    mesh = plsc.VectorSubcoreMesh(core_axis_name="c", subcore_axis_name="s")
    n = b_dim * l_dim
    per_worker = n // _NUM_WORKERS
    nc = per_worker // _CHUNK
    assert nc % 2 == 0 and nc >= 4
    assert b_dim % _CHUNK == 0 and (b_dim & (b_dim - 1)) == 0
    b_shift = b_dim.bit_length() - 1

    @functools.partial(
        pl.kernel,
        mesh=mesh,
        out_type=jax.ShapeDtypeStruct((l_dim, d, b_dim), jnp.float32),
        scratch_types=[
            pltpu.VMEM((per_worker,), jnp.int32),
            [pltpu.VMEM((_CHUNK, d), jnp.float32) for _ in range(2)],
            [pltpu.VMEM((d, _CHUNK + 1), jnp.float32) for _ in range(2)],
            [pltpu.SemaphoreType.DMA for _ in range(2)],
            [pltpu.SemaphoreType.DMA for _ in range(2)],
        ],
        compiler_params=pltpu.CompilerParams(
            use_tc_tiling_on_sc=False, needs_layout_passes=False),
    )
    def body(idx_hbm, table_hbm, out_hbm, idx_v, rows_v, t_v, gsem, wsem):
        wid = lax.axis_index("s") * 2 + lax.axis_index("c")
        base = wid * per_worker
        pltpu.sync_copy(idx_hbm.at[pl.ds(base, per_worker)], idx_v)

        def start_gather(ci, b):
            off = pl.multiple_of(ci * _CHUNK, 8)
            pltpu.async_copy(
                table_hbm.at[idx_v.at[pl.ds(off, _CHUNK)]], rows_v[b], gsem[b])

        def wait_gather(b):
            pltpu.make_async_copy(
                table_hbm.at[idx_v.at[pl.ds(0, _CHUNK)]], rows_v[b],
                gsem[b]).wait()

        def wait_writeback(b):
            pltpu.make_async_copy(
                t_v[b].at[:, pl.ds(0, _CHUNK)],
                out_hbm.at[0, :, pl.ds(0, _CHUNK)], wsem[b]).wait()

        start_gather(0, 0)
        start_gather(1, 1)

        @pl.loop(0, nc // 2)
        def pair_loop(p):
            for b in range(2):
                ci = p * 2 + b
                wait_gather(b)

                @pl.when(p > 0)
                def _():
                    wait_writeback(b)

                jlo = lax.iota(jnp.int32, _LANES)
                jhi = jlo + _LANES

                def transpose4(r4, carry, rows=rows_v[b], t=t_v[b]):
                    for k in range(4):
                        r = r4 * 4 + k
                        rvec = jnp.full((_LANES,), r, jnp.int32)
                        plsc.store_scatter(
                            t, [jlo, rvec], rows[r, pl.ds(0, _LANES)] * scale)
                        plsc.store_scatter(
                            t, [jhi, rvec],
                            rows[r, pl.ds(_LANES, _LANES)] * scale)
                    return carry

                lax.fori_loop(0, _CHUNK // 4, transpose4, 0)
                gpos = base + ci * _CHUNK
                l_i = lax.shift_right_logical(gpos, b_shift)
                b0 = pl.multiple_of(gpos & (b_dim - 1), _CHUNK)
                pltpu.async_copy(
                    t_v[b].at[:, pl.ds(0, _CHUNK)],
                    out_hbm.at[l_i, :, pl.ds(b0, _CHUNK)], wsem[b])

                @pl.when(ci + 2 < nc)
                def _():
                    start_gather(ci + 2, b)

        for b in range(2):
            wait_writeback(b)

    return body(flat_idx, table)


def kernel(tokens, table):
    b_dim, l_dim = tokens.shape
    v, d = table.shape
    flat_idx = tokens.T.reshape(b_dim * l_dim).astype(jnp.int32)
    out_nat = _emb_lookup(
        flat_idx,
        table,
        b_dim=b_dim,
        l_dim=l_dim,
        d=d,
        scale=math.sqrt(d),
    )
    return jnp.transpose(out_nat, (2, 0, 1))


# speedup vs baseline: 1.2947x; 1.0053x over previous
import functools
import math

import jax
import jax.numpy as jnp
from jax import lax
from jax.experimental import pallas as pl
from jax.experimental.pallas import tpu as pltpu
from jax.experimental.pallas import tpu_sc as plsc

_NUM_WORKERS = 32
_CHUNK = 512
_LANES = 16


def _emb_lookup(flat_idx, table, *, b_dim, l_dim, d, scale):
    mesh = plsc.VectorSubcoreMesh(core_axis_name="c", subcore_axis_name="s")
    n = b_dim * l_dim
    per_worker = n // _NUM_WORKERS
    nc = per_worker // _CHUNK
    assert nc % 2 == 0 and nc >= 4
    assert b_dim % _CHUNK == 0 and (b_dim & (b_dim - 1)) == 0
    b_shift = b_dim.bit_length() - 1

    @functools.partial(
        pl.kernel,
        mesh=mesh,
        out_type=jax.ShapeDtypeStruct((l_dim, d, b_dim), jnp.float32),
        scratch_types=[
            pltpu.VMEM((per_worker,), jnp.int32),
            [pltpu.VMEM((_CHUNK, d), jnp.float32) for _ in range(2)],
            [pltpu.VMEM((d, _CHUNK + 1), jnp.float32) for _ in range(2)],
            [pltpu.SemaphoreType.DMA for _ in range(2)],
            [pltpu.SemaphoreType.DMA for _ in range(2)],
        ],
        compiler_params=pltpu.CompilerParams(
            use_tc_tiling_on_sc=False, needs_layout_passes=False),
    )
    def body(idx_hbm, table_hbm, out_hbm, idx_v, rows_v, t_v, gsem, wsem):
        wid = lax.axis_index("s") * 2 + lax.axis_index("c")
        base = wid * per_worker
        pltpu.sync_copy(idx_hbm.at[pl.ds(base, per_worker)], idx_v)

        def start_gather(ci, b):
            off = pl.multiple_of(ci * _CHUNK, 8)
            pltpu.async_copy(
                table_hbm.at[idx_v.at[pl.ds(off, _CHUNK)]], rows_v[b], gsem[b])

        def wait_gather(b):
            pltpu.make_async_copy(
                table_hbm.at[idx_v.at[pl.ds(0, _CHUNK)]], rows_v[b],
                gsem[b]).wait()

        def wait_writeback(b):
            pltpu.make_async_copy(
                t_v[b].at[:, pl.ds(0, _CHUNK)],
                out_hbm.at[0, :, pl.ds(0, _CHUNK)], wsem[b]).wait()

        start_gather(0, 0)
        start_gather(1, 1)

        @pl.loop(0, nc // 2)
        def pair_loop(p):
            for b in range(2):
                ci = p * 2 + b
                wait_gather(b)

                @pl.when(p > 0)
                def _():
                    wait_writeback(b)

                jlo = lax.iota(jnp.int32, _LANES)
                jhi = jlo + _LANES

                def transpose4(r4, carry, rows=rows_v[b], t=t_v[b]):
                    for k in range(8):
                        r = r4 * 8 + k
                        rvec = jnp.full((_LANES,), r, jnp.int32)
                        plsc.store_scatter(
                            t, [jlo, rvec], rows[r, pl.ds(0, _LANES)] * scale)
                        plsc.store_scatter(
                            t, [jhi, rvec],
                            rows[r, pl.ds(_LANES, _LANES)] * scale)
                    return carry

                lax.fori_loop(0, _CHUNK // 8, transpose4, 0)
                gpos = base + ci * _CHUNK
                l_i = lax.shift_right_logical(gpos, b_shift)
                b0 = pl.multiple_of(gpos & (b_dim - 1), _CHUNK)
                pltpu.async_copy(
                    t_v[b].at[:, pl.ds(0, _CHUNK)],
                    out_hbm.at[l_i, :, pl.ds(b0, _CHUNK)], wsem[b])

                @pl.when(ci + 2 < nc)
                def _():
                    start_gather(ci + 2, b)

        for b in range(2):
            wait_writeback(b)

    return body(flat_idx, table)


def kernel(tokens, table):
    b_dim, l_dim = tokens.shape
    v, d = table.shape
    flat_idx = tokens.T.reshape(b_dim * l_dim).astype(jnp.int32)
    out_nat = _emb_lookup(
        flat_idx,
        table,
        b_dim=b_dim,
        l_dim=l_dim,
        d=d,
        scale=math.sqrt(d),
    )
    return jnp.transpose(out_nat, (2, 0, 1))
